# Initial kernel scaffold; baseline (speedup 1.0000x reference)
import jax
import jax.numpy as jnp
from jax.experimental import pallas as pl


def kernel(x, edge_index, alpha, k_ricci, e_poinc, params):
    raise NotImplementedError("write your pallas kernel here")


# speedup vs baseline: 4.0004x; 4.0004x over previous
import functools

import jax
import jax.numpy as jnp
from jax import lax
from jax.experimental import pallas as pl
from jax.experimental.pallas import tpu as pltpu
from jax.experimental.pallas import tpu_sc as plsc

N = 10000
E = 320000
D = 128
NSUB = 16
NCORE = 2
NW = NCORE * NSUB
K = 128
NCHUNK = E // K
CPW = (NCHUNK + NW - 1) // NW
RPS = 624
TAIL = N - RPS * NSUB
TAIL0 = RPS * NSUB
SR = 104

_MESH = plsc.VectorSubcoreMesh(core_axis_name="c", subcore_axis_name="s")


def _per_sub_rows(sid, fn):
    for off in range(0, RPS, SR):
        fn(sid * RPS + off, SR)

    @pl.when(sid == NSUB - 1)
    def _():
        fn(TAIL0, TAIL)


def _zero_shared(sid, z_hbm, stage_v, acc_sh):
    def zero(r0, nr):
        pltpu.sync_copy(z_hbm.at[pl.ds(r0, nr)], stage_v.at[pl.ds(0, nr)])
        pltpu.sync_copy(stage_v.at[pl.ds(0, nr)], acc_sh.at[pl.ds(r0, nr)])

    _per_sub_rows(sid, zero)


def _write_shared(sid, acc_sh, stage_v, out_hbm, cid):
    def wb(r0, nr):
        pltpu.sync_copy(acc_sh.at[pl.ds(r0, nr)], stage_v.at[pl.ds(0, nr)])
        pltpu.sync_copy(stage_v.at[pl.ds(0, nr)],
                        out_hbm.at[pl.ds(cid * N + r0, nr)])

    _per_sub_rows(sid, wb)


def _edge_weights(kr, W1, W2, b2):
    BE = 1280

    def body(kr_ref, w1_ref, w2_ref, b_ref, o_ref):
        h = jnp.dot(kr_ref[...], w1_ref[...], preferred_element_type=jnp.float32)
        h = jnp.where(h >= 0, h, 0.2 * h)
        ow = jnp.dot(h, w2_ref[...], preferred_element_type=jnp.float32) + b_ref[...]
        o_ref[...] = jnp.exp(ow)

    return pl.pallas_call(
        body,
        grid=(E // BE,),
        in_specs=[
            pl.BlockSpec((BE, kr.shape[1]), lambda i: (i, 0)),
            pl.BlockSpec(W1.shape, lambda i: (0, 0)),
            pl.BlockSpec(W2.shape, lambda i: (0, 0)),
            pl.BlockSpec((1, D), lambda i: (0, 0)),
        ],
        out_specs=pl.BlockSpec((BE, D), lambda i: (i, 0)),
        out_shape=jax.ShapeDtypeStruct((E, D), jnp.float32),
    )(kr, W1, W2, b2.reshape(1, D))


def _sc_segment_sum_src(e, src2, dst2):
    zeros = jnp.zeros((N, D), jnp.float32)
    zeros16 = jnp.zeros((N, 16), jnp.float32)
    ones16 = jnp.ones((K, 16), jnp.float32)

    @functools.partial(
        pl.kernel,
        out_type=[jax.ShapeDtypeStruct((NCORE * N, D), jnp.float32),
                  jax.ShapeDtypeStruct((NCORE * N, 16), jnp.float32)],
        mesh=_MESH,
        scratch_types=[
            pltpu.VMEM_SHARED((N, D), jnp.float32),
            pltpu.VMEM_SHARED((N, 16), jnp.float32),
            pltpu.VMEM((K, D), jnp.float32),
            pltpu.VMEM((1, K), jnp.int32),
            pltpu.VMEM((1, K), jnp.int32),
            pltpu.VMEM((K, 16), jnp.float32),
        ],
    )
    def k(e_hbm, src_hbm, dst_hbm, z_hbm, z16_hbm, ones_hbm,
          s_out, cnt_out, acc_sh, cnt_sh, e_v, si_v, di_v, ones_v):
        cid = lax.axis_index("c")
        sid = lax.axis_index("s")
        wid = sid * NCORE + cid
        _zero_shared(sid, z_hbm, e_v, acc_sh)
        _zero_shared(sid, z16_hbm, ones_v, cnt_sh)
        pltpu.sync_copy(ones_hbm, ones_v)
        plsc.subcore_barrier()

        @pl.loop(0, CPW)
        def _(i):
            c = wid + i * NW

            @pl.when(c < NCHUNK)
            def _():
                base = c * K
                pltpu.sync_copy(src_hbm.at[pl.ds(c, 1)], si_v)
                pltpu.sync_copy(dst_hbm.at[pl.ds(c, 1)], di_v)
                pltpu.sync_copy(e_hbm.at[pl.ds(base, K)], e_v)
                pltpu.sync_copy(e_v, acc_sh.at[si_v.at[0]], add=True)
                pltpu.sync_copy(ones_v, cnt_sh.at[di_v.at[0]], add=True)

        plsc.subcore_barrier()
        _write_shared(sid, acc_sh, e_v, s_out, cid)
        _write_shared(sid, cnt_sh, ones_v, cnt_out, cid)

    return k(e, src2, dst2, zeros, zeros16, ones16)


def _sc_count_dst(dst2):
    zeros = jnp.zeros((N, D), jnp.float32)
    ones = jnp.ones((K, D), jnp.float32)

    @functools.partial(
        pl.kernel,
        out_type=jax.ShapeDtypeStruct((NCORE * N, D), jnp.float32),
        mesh=_MESH,
        scratch_types=[
            pltpu.VMEM_SHARED((N, D), jnp.float32),
            pltpu.VMEM((K, D), jnp.float32),
            pltpu.VMEM((1, K), jnp.int32),
        ],
    )
    def k(dst_hbm, z_hbm, ones_hbm, cnt_out, cnt_sh, ones_v, di_v):
        cid = lax.axis_index("c")
        sid = lax.axis_index("s")
        wid = sid * NCORE + cid
        _zero_shared(sid, z_hbm, ones_v, cnt_sh)
        pltpu.sync_copy(ones_hbm, ones_v)
        plsc.subcore_barrier()

        @pl.loop(0, CPW)
        def _(i):
            c = wid + i * NW

            @pl.when(c < NCHUNK)
            def _():
                pltpu.sync_copy(dst_hbm.at[pl.ds(c, 1)], di_v)
                pltpu.sync_copy(ones_v, cnt_sh.at[di_v.at[0]], add=True)

        plsc.subcore_barrier()
        _write_shared(sid, cnt_sh, ones_v, cnt_out, cid)

    return k(dst2, zeros, ones)


def _sc_segment_sum_src_nocnt(e, src2):
    zeros = jnp.zeros((N, D), jnp.float32)

    @functools.partial(
        pl.kernel,
        out_type=jax.ShapeDtypeStruct((NCORE * N, D), jnp.float32),
        mesh=_MESH,
        scratch_types=[
            pltpu.VMEM_SHARED((N, D), jnp.float32),
            pltpu.VMEM((K, D), jnp.float32),
            pltpu.VMEM((1, K), jnp.int32),
        ],
    )
    def k(e_hbm, src_hbm, z_hbm, s_out, acc_sh, e_v, si_v):
        cid = lax.axis_index("c")
        sid = lax.axis_index("s")
        wid = sid * NCORE + cid
        _zero_shared(sid, z_hbm, e_v, acc_sh)
        plsc.subcore_barrier()

        @pl.loop(0, CPW)
        def _(i):
            c = wid + i * NW

            @pl.when(c < NCHUNK)
            def _():
                base = c * K
                pltpu.sync_copy(src_hbm.at[pl.ds(c, 1)], si_v)
                pltpu.sync_copy(e_hbm.at[pl.ds(base, K)], e_v)
                pltpu.sync_copy(e_v, acc_sh.at[si_v.at[0]], add=True)

        plsc.subcore_barrier()
        _write_shared(sid, acc_sh, e_v, s_out, cid)

    return k(e, src2, zeros)


def _sc_gather_mul_scatter(e, g, src2, dst2):
    zeros = jnp.zeros((N, D), jnp.float32)

    @functools.partial(
        pl.kernel,
        out_type=jax.ShapeDtypeStruct((NCORE * N, D), jnp.float32),
        mesh=_MESH,
        scratch_types=[
            pltpu.VMEM_SHARED((N, D), jnp.float32),
            pltpu.VMEM((K, D), jnp.float32),
            pltpu.VMEM((K, D), jnp.float32),
            pltpu.VMEM((1, K), jnp.int32),
            pltpu.VMEM((1, K), jnp.int32),
        ],
    )
    def k(e_hbm, g_hbm, src_hbm, dst_hbm, z_hbm, acc_out,
          acc_sh, e_v, g_v, si_v, di_v):
        cid = lax.axis_index("c")
        sid = lax.axis_index("s")
        wid = sid * NCORE + cid
        _zero_shared(sid, z_hbm, e_v, acc_sh)
        plsc.subcore_barrier()

        @pl.loop(0, CPW)
        def _(i):
            c = wid + i * NW

            @pl.when(c < NCHUNK)
            def _():
                base = c * K
                pltpu.sync_copy(src_hbm.at[pl.ds(c, 1)], si_v)
                pltpu.sync_copy(dst_hbm.at[pl.ds(c, 1)], di_v)
                pltpu.sync_copy(e_hbm.at[pl.ds(base, K)], e_v)
                pltpu.sync_copy(g_hbm.at[si_v.at[0]], g_v)

                @pl.loop(0, K)
                def _(r):
                    for j in range(D // 16):
                        sl = pl.ds(j * 16, 16)
                        e_v[r, sl] = e_v[r, sl] * g_v[r, sl]

                pltpu.sync_copy(e_v, acc_sh.at[di_v.at[0]], add=True)

        plsc.subcore_barrier()
        _write_shared(sid, acc_sh, e_v, acc_out, cid)

    return k(e, g, src2, dst2, zeros)


def _gdiv(x, s):
    BN = 2000
    NB = N // BN

    def body(x_ref, s0_ref, s1_ref, o_ref):
        o_ref[...] = x_ref[...] / (s0_ref[...] + s1_ref[...] + 1e-16)

    return pl.pallas_call(
        body,
        grid=(NB,),
        in_specs=[
            pl.BlockSpec((BN, D), lambda i: (i, 0)),
            pl.BlockSpec((BN, D), lambda i: (i, 0)),
            pl.BlockSpec((BN, D), lambda i: (i + NB, 0)),
        ],
        out_specs=pl.BlockSpec((BN, D), lambda i: (i, 0)),
        out_shape=jax.ShapeDtypeStruct((N, D), jnp.float32),
    )(x, s, s)


def _finish(acc, cnt, x, ep, p, alpha, last):
    BN = 2000
    NB = N // BN
    Dout = p["lW"].shape[1]

    def body(a0_ref, a1_ref, c0_ref, c1_ref, x_ref, ep_ref, al_ref,
             lW_ref, lb_ref, pW1_ref, pW2_ref, pb2_ref, rW_ref, o_ref):
        a = a0_ref[...] + a1_ref[...]
        c = c0_ref[:, 0:1] + c1_ref[:, 0:1]
        agg = a / jnp.maximum(c, 1.0)
        out = jnp.dot(agg, lW_ref[...], preferred_element_type=jnp.float32) + lb_ref[...]
        pw = jnp.dot(ep_ref[...], pW1_ref[...], preferred_element_type=jnp.float32)
        pw = jnp.where(pw >= 0, pw, 0.2 * pw)
        pw = jnp.dot(pw, pW2_ref[...], preferred_element_type=jnp.float32) + pb2_ref[...]
        pw = jnp.where(pw >= 0, pw, 0.01 * pw)
        out = out + al_ref[0, 0] * pw
        out = out + jnp.dot(x_ref[...], rW_ref[...], preferred_element_type=jnp.float32)
        if last:
            m = jnp.max(out, axis=1, keepdims=True)
            z = out - m
            out = z - jnp.log(jnp.sum(jnp.exp(z), axis=1, keepdims=True))
        else:
            out = jnp.maximum(out, 0.0)
        o_ref[...] = out

    H2 = p["pW1"].shape[1]
    return pl.pallas_call(
        body,
        grid=(NB,),
        in_specs=[
            pl.BlockSpec((BN, D), lambda i: (i, 0)),
            pl.BlockSpec((BN, D), lambda i: (i + NB, 0)),
            pl.BlockSpec((BN, D), lambda i: (i, 0)),
            pl.BlockSpec((BN, D), lambda i: (i + NB, 0)),
            pl.BlockSpec((BN, D), lambda i: (i, 0)),
            pl.BlockSpec((BN, ep.shape[1]), lambda i: (i, 0)),
            pl.BlockSpec((1, 1), lambda i: (0, 0)),
            pl.BlockSpec(p["lW"].shape, lambda i: (0, 0)),
            pl.BlockSpec((1, Dout), lambda i: (0, 0)),
            pl.BlockSpec(p["pW1"].shape, lambda i: (0, 0)),
            pl.BlockSpec(p["pW2"].shape, lambda i: (0, 0)),
            pl.BlockSpec((1, H2), lambda i: (0, 0)),
            pl.BlockSpec(p["rW"].shape, lambda i: (0, 0)),
        ],
        out_specs=pl.BlockSpec((BN, Dout), lambda i: (i, 0)),
        out_shape=jax.ShapeDtypeStruct((N, Dout), jnp.float32),
    )(acc, acc, cnt, cnt, x, ep,
      jnp.asarray(alpha, jnp.float32).reshape(1, 1),
      p["lW"], p["lb"].reshape(1, Dout),
      p["pW1"], p["pW2"], p["pb2"].reshape(1, H2), p["rW"])


def kernel(x, edge_index, alpha, k_ricci, e_poinc, params):
    src2 = edge_index[0].reshape(NCHUNK, K)
    dst2 = edge_index[1].reshape(NCHUNK, K)
    p1, p2 = params["l1"], params["l2"]

    e1 = _edge_weights(k_ricci, p1["hW1"], p1["hW2"], p1["hb2"])
    s1 = _sc_segment_sum_src_nocnt(e1, src2)
    cnt = _sc_count_dst(dst2)
    g1 = _gdiv(x, s1)
    acc1 = _sc_gather_mul_scatter(e1, g1, src2, dst2)
    h = _finish(acc1, cnt, x, e_poinc, p1, alpha, last=False)

    e2 = _edge_weights(k_ricci, p2["hW1"], p2["hW2"], p2["hb2"])
    s2 = _sc_segment_sum_src_nocnt(e2, src2)
    g2 = _gdiv(h, s2)
    acc2 = _sc_gather_mul_scatter(e2, g2, src2, dst2)
    return _finish(acc2, cnt, h, e_poinc, p2, alpha, last=True)


# speedup vs baseline: 4.6039x; 1.1509x over previous
import functools

import jax
import jax.numpy as jnp
from jax import lax
from jax.experimental import pallas as pl
from jax.experimental.pallas import tpu as pltpu
from jax.experimental.pallas import tpu_sc as plsc

N = 10000
E = 320000
D = 128
NSUB = 16
NCORE = 2
NW = NCORE * NSUB
K = 128
NCHUNK = E // K
CPW = (NCHUNK + NW - 1) // NW
RPS = 624
TAIL = N - RPS * NSUB
TAIL0 = RPS * NSUB
SR = 104
KB = 64
NCHUNK_B = E // KB
CPW_B = (NCHUNK_B + NW - 1) // NW
HALF_B = (CPW_B + 1) // 2

_MESH = plsc.VectorSubcoreMesh(core_axis_name="c", subcore_axis_name="s")


def _per_sub_rows(sid, fn):
    for off in range(0, RPS, SR):
        fn(sid * RPS + off, SR)

    @pl.when(sid == NSUB - 1)
    def _():
        fn(TAIL0, TAIL)


def _zero_shared(sid, z_hbm, stage_v, acc_sh):
    def zero(r0, nr):
        pltpu.sync_copy(z_hbm.at[pl.ds(r0, nr)], stage_v.at[pl.ds(0, nr)])
        pltpu.sync_copy(stage_v.at[pl.ds(0, nr)], acc_sh.at[pl.ds(r0, nr)])

    _per_sub_rows(sid, zero)


def _write_shared(sid, acc_sh, stage_v, out_hbm, cid):
    def wb(r0, nr):
        pltpu.sync_copy(acc_sh.at[pl.ds(r0, nr)], stage_v.at[pl.ds(0, nr)])
        pltpu.sync_copy(stage_v.at[pl.ds(0, nr)],
                        out_hbm.at[pl.ds(cid * N + r0, nr)])

    _per_sub_rows(sid, wb)


def _edge_weights(kr, W1, W2, b2):
    BE = 1280

    def body(kr_ref, w1_ref, w2_ref, b_ref, o_ref):
        h = jnp.dot(kr_ref[...], w1_ref[...], preferred_element_type=jnp.float32)
        h = jnp.where(h >= 0, h, 0.2 * h)
        ow = jnp.dot(h, w2_ref[...], preferred_element_type=jnp.float32) + b_ref[...]
        o_ref[...] = jnp.exp(ow)

    return pl.pallas_call(
        body,
        grid=(E // BE,),
        in_specs=[
            pl.BlockSpec((BE, kr.shape[1]), lambda i: (i, 0)),
            pl.BlockSpec(W1.shape, lambda i: (0, 0)),
            pl.BlockSpec(W2.shape, lambda i: (0, 0)),
            pl.BlockSpec((1, D), lambda i: (0, 0)),
        ],
        out_specs=pl.BlockSpec((BE, D), lambda i: (i, 0)),
        out_shape=jax.ShapeDtypeStruct((E, D), jnp.float32),
    )(kr, W1, W2, b2.reshape(1, D))


def _sc_count_dst(dst2):
    zeros = jnp.zeros((N, D), jnp.float32)
    ones = jnp.ones((K, D), jnp.float32)

    @functools.partial(
        pl.kernel,
        out_type=jax.ShapeDtypeStruct((NCORE * N, D), jnp.float32),
        mesh=_MESH,
        scratch_types=[
            pltpu.VMEM_SHARED((N, D), jnp.float32),
            pltpu.VMEM((K, D), jnp.float32),
            pltpu.VMEM((1, K), jnp.int32),
        ],
    )
    def k(dst_hbm, z_hbm, ones_hbm, cnt_out, cnt_sh, ones_v, di_v):
        cid = lax.axis_index("c")
        sid = lax.axis_index("s")
        wid = sid * NCORE + cid
        _zero_shared(sid, z_hbm, ones_v, cnt_sh)
        pltpu.sync_copy(ones_hbm, ones_v)
        plsc.subcore_barrier()

        @pl.loop(0, CPW)
        def _(i):
            c = wid + i * NW

            @pl.when(c < NCHUNK)
            def _():
                pltpu.sync_copy(dst_hbm.at[pl.ds(c, 1)], di_v)
                pltpu.sync_copy(ones_v, cnt_sh.at[di_v.at[0]], add=True)

        plsc.subcore_barrier()
        _write_shared(sid, cnt_sh, ones_v, cnt_out, cid)

    return k(dst2, zeros, ones)


def _sc_segment_sum_src_nocnt(e, src2):
    zeros = jnp.zeros((N, D), jnp.float32)

    @functools.partial(
        pl.kernel,
        out_type=jax.ShapeDtypeStruct((NCORE * N, D), jnp.float32),
        mesh=_MESH,
        scratch_types=[
            pltpu.VMEM_SHARED((N, D), jnp.float32),
            pltpu.VMEM((K, D), jnp.float32),
            pltpu.VMEM((1, K), jnp.int32),
        ],
    )
    def k(e_hbm, src_hbm, z_hbm, s_out, acc_sh, e_v, si_v):
        cid = lax.axis_index("c")
        sid = lax.axis_index("s")
        wid = sid * NCORE + cid
        _zero_shared(sid, z_hbm, e_v, acc_sh)
        plsc.subcore_barrier()

        @pl.loop(0, CPW)
        def _(i):
            c = wid + i * NW

            @pl.when(c < NCHUNK)
            def _():
                base = c * K
                pltpu.sync_copy(src_hbm.at[pl.ds(c, 1)], si_v)
                pltpu.sync_copy(e_hbm.at[pl.ds(base, K)], e_v)
                pltpu.sync_copy(e_v, acc_sh.at[si_v.at[0]], add=True)

        plsc.subcore_barrier()
        _write_shared(sid, acc_sh, e_v, s_out, cid)

    return k(e, src2, zeros)


def _sc_gather_mul_scatter(e, g, srcb, dstb):
    zeros = jnp.zeros((N, D), jnp.float32)

    @functools.partial(
        pl.kernel,
        out_type=jax.ShapeDtypeStruct((NCORE * N, D), jnp.float32),
        mesh=_MESH,
        scratch_types=[
            pltpu.VMEM_SHARED((N, D), jnp.float32),
            pltpu.VMEM((KB, D), jnp.float32),
            pltpu.VMEM((KB, D), jnp.float32),
            pltpu.VMEM((KB, D), jnp.float32),
            pltpu.VMEM((KB, D), jnp.float32),
            pltpu.VMEM((1, KB), jnp.int32),
            pltpu.VMEM((1, KB), jnp.int32),
            pltpu.VMEM((1, KB), jnp.int32),
            pltpu.VMEM((1, KB), jnp.int32),
            pltpu.SemaphoreType.DMA,
            pltpu.SemaphoreType.DMA,
        ],
    )
    def k(e_hbm, g_hbm, src_hbm, dst_hbm, z_hbm, acc_out,
          acc_sh, e_v0, e_v1, g_v0, g_v1, si_v0, si_v1, di_v0, di_v1,
          sem0, sem1):
        cid = lax.axis_index("c")
        sid = lax.axis_index("s")
        wid = sid * NCORE + cid
        _zero_shared(sid, z_hbm, e_v0, acc_sh)
        plsc.subcore_barrier()

        ebufs = (e_v0, e_v1)
        gbufs = (g_v0, g_v1)
        sibufs = (si_v0, si_v1)
        dibufs = (di_v0, di_v1)
        sems = (sem0, sem1)

        def issue(c, p):
            base = c * KB
            pltpu.sync_copy(src_hbm.at[pl.ds(c, 1)], sibufs[p])
            pltpu.sync_copy(dst_hbm.at[pl.ds(c, 1)], dibufs[p])
            pltpu.async_copy(e_hbm.at[pl.ds(base, KB)], ebufs[p], sems[p])
            pltpu.async_copy(g_hbm.at[sibufs[p].at[0]], gbufs[p], sems[p])

        def consume(c, p):
            base = c * KB
            pltpu.make_async_copy(e_hbm.at[pl.ds(base, KB)], ebufs[p],
                                  sems[p]).wait()
            pltpu.make_async_copy(g_hbm.at[sibufs[p].at[0]], gbufs[p],
                                  sems[p]).wait()

            @pl.loop(0, KB)
            def _(r):
                for j in range(D // 16):
                    sl = pl.ds(j * 16, 16)
                    ebufs[p][r, sl] = ebufs[p][r, sl] * gbufs[p][r, sl]

            pltpu.sync_copy(ebufs[p], acc_sh.at[dibufs[p].at[0]], add=True)

        def chunk(j):
            return wid + j * NW

        @pl.when(chunk(0) < NCHUNK_B)
        def _():
            issue(chunk(0), 0)

        @pl.loop(0, HALF_B)
        def _(i):
            j0 = 2 * i
            c0 = wid + j0 * NW
            c1 = c0 + NW
            c2 = c1 + NW

            @pl.when(c1 < NCHUNK_B)
            def _():
                issue(c1, 1)

            @pl.when(c0 < NCHUNK_B)
            def _():
                consume(c0, 0)

            @pl.when(c2 < NCHUNK_B)
            def _():
                issue(c2, 0)

            @pl.when(c1 < NCHUNK_B)
            def _():
                consume(c1, 1)

        plsc.subcore_barrier()
        _write_shared(sid, acc_sh, e_v0, acc_out, cid)

    return k(e, g, srcb, dstb, zeros)


def _gdiv(x, s):
    BN = 2000
    NB = N // BN

    def body(x_ref, s0_ref, s1_ref, o_ref):
        o_ref[...] = x_ref[...] / (s0_ref[...] + s1_ref[...] + 1e-16)

    return pl.pallas_call(
        body,
        grid=(NB,),
        in_specs=[
            pl.BlockSpec((BN, D), lambda i: (i, 0)),
            pl.BlockSpec((BN, D), lambda i: (i, 0)),
            pl.BlockSpec((BN, D), lambda i: (i + NB, 0)),
        ],
        out_specs=pl.BlockSpec((BN, D), lambda i: (i, 0)),
        out_shape=jax.ShapeDtypeStruct((N, D), jnp.float32),
    )(x, s, s)


def _finish(acc, cnt, x, ep, p, alpha, last):
    BN = 2000
    NB = N // BN
    Dout = p["lW"].shape[1]

    def body(a0_ref, a1_ref, c0_ref, c1_ref, x_ref, ep_ref, al_ref,
             lW_ref, lb_ref, pW1_ref, pW2_ref, pb2_ref, rW_ref, o_ref):
        a = a0_ref[...] + a1_ref[...]
        c = c0_ref[:, 0:1] + c1_ref[:, 0:1]
        agg = a / jnp.maximum(c, 1.0)
        out = jnp.dot(agg, lW_ref[...], preferred_element_type=jnp.float32) + lb_ref[...]
        pw = jnp.dot(ep_ref[...], pW1_ref[...], preferred_element_type=jnp.float32)
        pw = jnp.where(pw >= 0, pw, 0.2 * pw)
        pw = jnp.dot(pw, pW2_ref[...], preferred_element_type=jnp.float32) + pb2_ref[...]
        pw = jnp.where(pw >= 0, pw, 0.01 * pw)
        out = out + al_ref[0, 0] * pw
        out = out + jnp.dot(x_ref[...], rW_ref[...], preferred_element_type=jnp.float32)
        if last:
            m = jnp.max(out, axis=1, keepdims=True)
            z = out - m
            out = z - jnp.log(jnp.sum(jnp.exp(z), axis=1, keepdims=True))
        else:
            out = jnp.maximum(out, 0.0)
        o_ref[...] = out

    H2 = p["pW1"].shape[1]
    return pl.pallas_call(
        body,
        grid=(NB,),
        in_specs=[
            pl.BlockSpec((BN, D), lambda i: (i, 0)),
            pl.BlockSpec((BN, D), lambda i: (i + NB, 0)),
            pl.BlockSpec((BN, D), lambda i: (i, 0)),
            pl.BlockSpec((BN, D), lambda i: (i + NB, 0)),
            pl.BlockSpec((BN, D), lambda i: (i, 0)),
            pl.BlockSpec((BN, ep.shape[1]), lambda i: (i, 0)),
            pl.BlockSpec((1, 1), lambda i: (0, 0)),
            pl.BlockSpec(p["lW"].shape, lambda i: (0, 0)),
            pl.BlockSpec((1, Dout), lambda i: (0, 0)),
            pl.BlockSpec(p["pW1"].shape, lambda i: (0, 0)),
            pl.BlockSpec(p["pW2"].shape, lambda i: (0, 0)),
            pl.BlockSpec((1, H2), lambda i: (0, 0)),
            pl.BlockSpec(p["rW"].shape, lambda i: (0, 0)),
        ],
        out_specs=pl.BlockSpec((BN, Dout), lambda i: (i, 0)),
        out_shape=jax.ShapeDtypeStruct((N, Dout), jnp.float32),
    )(acc, acc, cnt, cnt, x, ep,
      jnp.asarray(alpha, jnp.float32).reshape(1, 1),
      p["lW"], p["lb"].reshape(1, Dout),
      p["pW1"], p["pW2"], p["pb2"].reshape(1, H2), p["rW"])


def kernel(x, edge_index, alpha, k_ricci, e_poinc, params):
    src2 = edge_index[0].reshape(NCHUNK, K)
    dst2 = edge_index[1].reshape(NCHUNK, K)
    srcb = edge_index[0].reshape(NCHUNK_B, KB)
    dstb = edge_index[1].reshape(NCHUNK_B, KB)
    p1, p2 = params["l1"], params["l2"]

    e1 = _edge_weights(k_ricci, p1["hW1"], p1["hW2"], p1["hb2"])
    s1 = _sc_segment_sum_src_nocnt(e1, src2)
    cnt = _sc_count_dst(dst2)
    g1 = _gdiv(x, s1)
    acc1 = _sc_gather_mul_scatter(e1, g1, srcb, dstb)
    h = _finish(acc1, cnt, x, e_poinc, p1, alpha, last=False)

    e2 = _edge_weights(k_ricci, p2["hW1"], p2["hW2"], p2["hb2"])
    s2 = _sc_segment_sum_src_nocnt(e2, src2)
    g2 = _gdiv(h, s2)
    acc2 = _sc_gather_mul_scatter(e2, g2, srcb, dstb)
    return _finish(acc2, cnt, h, e_poinc, p2, alpha, last=True)


# speedup vs baseline: 5.1707x; 1.1231x over previous
import functools

import jax
import jax.numpy as jnp
from jax import lax
from jax.experimental import pallas as pl
from jax.experimental.pallas import tpu as pltpu
from jax.experimental.pallas import tpu_sc as plsc

N = 10000
E = 320000
D = 128
NSUB = 16
NCORE = 2
NW = NCORE * NSUB
K = 128
NCHUNK = E // K
CPW = (NCHUNK + NW - 1) // NW
RPS = 624
TAIL = N - RPS * NSUB
TAIL0 = RPS * NSUB
SR = 104
KB = 64
NCHUNK_B = E // KB
CPW_B = (NCHUNK_B + NW - 1) // NW
HALF_B = (CPW_B + 1) // 2
HALF = (CPW + 1) // 2

_MESH = plsc.VectorSubcoreMesh(core_axis_name="c", subcore_axis_name="s")


def _per_sub_rows(sid, fn):
    for off in range(0, RPS, SR):
        fn(sid * RPS + off, SR)

    @pl.when(sid == NSUB - 1)
    def _():
        fn(TAIL0, TAIL)


def _zero_shared(sid, z_hbm, stage_v, acc_sh):
    def zero(r0, nr):
        pltpu.sync_copy(z_hbm.at[pl.ds(r0, nr)], stage_v.at[pl.ds(0, nr)])
        pltpu.sync_copy(stage_v.at[pl.ds(0, nr)], acc_sh.at[pl.ds(r0, nr)])

    _per_sub_rows(sid, zero)


def _write_shared(sid, acc_sh, stage_v, out_hbm, cid):
    def wb(r0, nr):
        pltpu.sync_copy(acc_sh.at[pl.ds(r0, nr)], stage_v.at[pl.ds(0, nr)])
        pltpu.sync_copy(stage_v.at[pl.ds(0, nr)],
                        out_hbm.at[pl.ds(cid * N + r0, nr)])

    _per_sub_rows(sid, wb)


def _edge_weights(kr, W1, W2, b2):
    BE = 1280

    def body(kr_ref, w1_ref, w2_ref, b_ref, o_ref):
        h = jnp.dot(kr_ref[...], w1_ref[...], preferred_element_type=jnp.float32)
        h = jnp.where(h >= 0, h, 0.2 * h)
        ow = jnp.dot(h, w2_ref[...], preferred_element_type=jnp.float32) + b_ref[...]
        o_ref[...] = jnp.exp(ow)

    return pl.pallas_call(
        body,
        grid=(E // BE,),
        in_specs=[
            pl.BlockSpec((BE, kr.shape[1]), lambda i: (i, 0)),
            pl.BlockSpec(W1.shape, lambda i: (0, 0)),
            pl.BlockSpec(W2.shape, lambda i: (0, 0)),
            pl.BlockSpec((1, D), lambda i: (0, 0)),
        ],
        out_specs=pl.BlockSpec((BE, D), lambda i: (i, 0)),
        out_shape=jax.ShapeDtypeStruct((E, D), jnp.float32),
    )(kr, W1, W2, b2.reshape(1, D))


def _sc_count_dst(dst2):
    zeros = jnp.zeros((N, D), jnp.float32)
    ones = jnp.ones((K, D), jnp.float32)

    @functools.partial(
        pl.kernel,
        out_type=jax.ShapeDtypeStruct((NCORE * N, D), jnp.float32),
        mesh=_MESH,
        scratch_types=[
            pltpu.VMEM_SHARED((N, D), jnp.float32),
            pltpu.VMEM((K, D), jnp.float32),
            pltpu.VMEM((1, K), jnp.int32),
        ],
    )
    def k(dst_hbm, z_hbm, ones_hbm, cnt_out, cnt_sh, ones_v, di_v):
        cid = lax.axis_index("c")
        sid = lax.axis_index("s")
        wid = sid * NCORE + cid
        _zero_shared(sid, z_hbm, ones_v, cnt_sh)
        pltpu.sync_copy(ones_hbm, ones_v)
        plsc.subcore_barrier()

        @pl.loop(0, CPW)
        def _(i):
            c = wid + i * NW

            @pl.when(c < NCHUNK)
            def _():
                pltpu.sync_copy(dst_hbm.at[pl.ds(c, 1)], di_v)
                pltpu.sync_copy(ones_v, cnt_sh.at[di_v.at[0]], add=True)

        plsc.subcore_barrier()
        _write_shared(sid, cnt_sh, ones_v, cnt_out, cid)

    return k(dst2, zeros, ones)


def _sc_segment_sum_src_nocnt(e, src2):
    zeros = jnp.zeros((N, D), jnp.float32)

    @functools.partial(
        pl.kernel,
        out_type=jax.ShapeDtypeStruct((NCORE * N, D), jnp.float32),
        mesh=_MESH,
        scratch_types=[
            pltpu.VMEM_SHARED((N, D), jnp.float32),
            pltpu.VMEM((K, D), jnp.float32),
            pltpu.VMEM((K, D), jnp.float32),
            pltpu.VMEM((1, K), jnp.int32),
            pltpu.VMEM((1, K), jnp.int32),
            pltpu.SemaphoreType.DMA,
            pltpu.SemaphoreType.DMA,
        ],
    )
    def k(e_hbm, src_hbm, z_hbm, s_out,
          acc_sh, e_v0, e_v1, si_v0, si_v1, sem0, sem1):
        cid = lax.axis_index("c")
        sid = lax.axis_index("s")
        wid = sid * NCORE + cid
        _zero_shared(sid, z_hbm, e_v0, acc_sh)
        plsc.subcore_barrier()

        ebufs = (e_v0, e_v1)
        sibufs = (si_v0, si_v1)
        sems = (sem0, sem1)

        def issue(c, p):
            pltpu.sync_copy(src_hbm.at[pl.ds(c, 1)], sibufs[p])
            pltpu.async_copy(e_hbm.at[pl.ds(c * K, K)], ebufs[p], sems[p])

        def consume(c, p):
            pltpu.make_async_copy(e_hbm.at[pl.ds(c * K, K)], ebufs[p],
                                  sems[p]).wait()
            pltpu.sync_copy(ebufs[p], acc_sh.at[sibufs[p].at[0]], add=True)

        @pl.when(wid < NCHUNK)
        def _():
            issue(wid, 0)

        @pl.loop(0, HALF)
        def _(i):
            c0 = wid + 2 * i * NW
            c1 = c0 + NW
            c2 = c1 + NW

            @pl.when(c1 < NCHUNK)
            def _():
                issue(c1, 1)

            @pl.when(c0 < NCHUNK)
            def _():
                consume(c0, 0)

            @pl.when(c2 < NCHUNK)
            def _():
                issue(c2, 0)

            @pl.when(c1 < NCHUNK)
            def _():
                consume(c1, 1)

        plsc.subcore_barrier()
        _write_shared(sid, acc_sh, e_v0, s_out, cid)

    return k(e, src2, zeros)


def _sc_gather_mul_scatter(e, g, srcb, dstb):
    zeros = jnp.zeros((N, D), jnp.float32)

    @functools.partial(
        pl.kernel,
        out_type=jax.ShapeDtypeStruct((NCORE * N, D), jnp.float32),
        mesh=_MESH,
        scratch_types=[
            pltpu.VMEM_SHARED((N, D), jnp.float32),
            pltpu.VMEM((KB, D), jnp.float32),
            pltpu.VMEM((KB, D), jnp.float32),
            pltpu.VMEM((KB, D), jnp.float32),
            pltpu.VMEM((KB, D), jnp.float32),
            pltpu.VMEM((1, KB), jnp.int32),
            pltpu.VMEM((1, KB), jnp.int32),
            pltpu.VMEM((1, KB), jnp.int32),
            pltpu.VMEM((1, KB), jnp.int32),
            pltpu.SemaphoreType.DMA,
            pltpu.SemaphoreType.DMA,
        ],
    )
    def k(e_hbm, g_hbm, src_hbm, dst_hbm, z_hbm, acc_out,
          acc_sh, e_v0, e_v1, g_v0, g_v1, si_v0, si_v1, di_v0, di_v1,
          sem0, sem1):
        cid = lax.axis_index("c")
        sid = lax.axis_index("s")
        wid = sid * NCORE + cid
        _zero_shared(sid, z_hbm, e_v0, acc_sh)
        plsc.subcore_barrier()

        ebufs = (e_v0, e_v1)
        gbufs = (g_v0, g_v1)
        sibufs = (si_v0, si_v1)
        dibufs = (di_v0, di_v1)
        sems = (sem0, sem1)

        def issue(c, p):
            base = c * KB
            pltpu.sync_copy(src_hbm.at[pl.ds(c, 1)], sibufs[p])
            pltpu.sync_copy(dst_hbm.at[pl.ds(c, 1)], dibufs[p])
            pltpu.async_copy(e_hbm.at[pl.ds(base, KB)], ebufs[p], sems[p])
            pltpu.async_copy(g_hbm.at[sibufs[p].at[0]], gbufs[p], sems[p])

        def consume(c, p):
            base = c * KB
            pltpu.make_async_copy(e_hbm.at[pl.ds(base, KB)], ebufs[p],
                                  sems[p]).wait()
            pltpu.make_async_copy(g_hbm.at[sibufs[p].at[0]], gbufs[p],
                                  sems[p]).wait()

            @pl.loop(0, KB)
            def _(r):
                for j in range(D // 16):
                    sl = pl.ds(j * 16, 16)
                    ebufs[p][r, sl] = ebufs[p][r, sl] * gbufs[p][r, sl]

            pltpu.sync_copy(ebufs[p], acc_sh.at[dibufs[p].at[0]], add=True)

        def chunk(j):
            return wid + j * NW

        @pl.when(chunk(0) < NCHUNK_B)
        def _():
            issue(chunk(0), 0)

        @pl.loop(0, HALF_B)
        def _(i):
            j0 = 2 * i
            c0 = wid + j0 * NW
            c1 = c0 + NW
            c2 = c1 + NW

            @pl.when(c1 < NCHUNK_B)
            def _():
                issue(c1, 1)

            @pl.when(c0 < NCHUNK_B)
            def _():
                consume(c0, 0)

            @pl.when(c2 < NCHUNK_B)
            def _():
                issue(c2, 0)

            @pl.when(c1 < NCHUNK_B)
            def _():
                consume(c1, 1)

        plsc.subcore_barrier()
        _write_shared(sid, acc_sh, e_v0, acc_out, cid)

    return k(e, g, srcb, dstb, zeros)


def _gdiv(x, s):
    BN = 2000
    NB = N // BN

    def body(x_ref, s0_ref, s1_ref, o_ref):
        o_ref[...] = x_ref[...] / (s0_ref[...] + s1_ref[...] + 1e-16)

    return pl.pallas_call(
        body,
        grid=(NB,),
        in_specs=[
            pl.BlockSpec((BN, D), lambda i: (i, 0)),
            pl.BlockSpec((BN, D), lambda i: (i, 0)),
            pl.BlockSpec((BN, D), lambda i: (i + NB, 0)),
        ],
        out_specs=pl.BlockSpec((BN, D), lambda i: (i, 0)),
        out_shape=jax.ShapeDtypeStruct((N, D), jnp.float32),
    )(x, s, s)


def _finish(acc, cnt, x, ep, p, alpha, last):
    BN = 2000
    NB = N // BN
    Dout = p["lW"].shape[1]

    def body(a0_ref, a1_ref, c0_ref, c1_ref, x_ref, ep_ref, al_ref,
             lW_ref, lb_ref, pW1_ref, pW2_ref, pb2_ref, rW_ref, o_ref):
        a = a0_ref[...] + a1_ref[...]
        c = c0_ref[:, 0:1] + c1_ref[:, 0:1]
        agg = a / jnp.maximum(c, 1.0)
        out = jnp.dot(agg, lW_ref[...], preferred_element_type=jnp.float32) + lb_ref[...]
        pw = jnp.dot(ep_ref[...], pW1_ref[...], preferred_element_type=jnp.float32)
        pw = jnp.where(pw >= 0, pw, 0.2 * pw)
        pw = jnp.dot(pw, pW2_ref[...], preferred_element_type=jnp.float32) + pb2_ref[...]
        pw = jnp.where(pw >= 0, pw, 0.01 * pw)
        out = out + al_ref[0, 0] * pw
        out = out + jnp.dot(x_ref[...], rW_ref[...], preferred_element_type=jnp.float32)
        if last:
            m = jnp.max(out, axis=1, keepdims=True)
            z = out - m
            out = z - jnp.log(jnp.sum(jnp.exp(z), axis=1, keepdims=True))
        else:
            out = jnp.maximum(out, 0.0)
        o_ref[...] = out

    H2 = p["pW1"].shape[1]
    return pl.pallas_call(
        body,
        grid=(NB,),
        in_specs=[
            pl.BlockSpec((BN, D), lambda i: (i, 0)),
            pl.BlockSpec((BN, D), lambda i: (i + NB, 0)),
            pl.BlockSpec((BN, D), lambda i: (i, 0)),
            pl.BlockSpec((BN, D), lambda i: (i + NB, 0)),
            pl.BlockSpec((BN, D), lambda i: (i, 0)),
            pl.BlockSpec((BN, ep.shape[1]), lambda i: (i, 0)),
            pl.BlockSpec((1, 1), lambda i: (0, 0)),
            pl.BlockSpec(p["lW"].shape, lambda i: (0, 0)),
            pl.BlockSpec((1, Dout), lambda i: (0, 0)),
            pl.BlockSpec(p["pW1"].shape, lambda i: (0, 0)),
            pl.BlockSpec(p["pW2"].shape, lambda i: (0, 0)),
            pl.BlockSpec((1, H2), lambda i: (0, 0)),
            pl.BlockSpec(p["rW"].shape, lambda i: (0, 0)),
        ],
        out_specs=pl.BlockSpec((BN, Dout), lambda i: (i, 0)),
        out_shape=jax.ShapeDtypeStruct((N, Dout), jnp.float32),
    )(acc, acc, cnt, cnt, x, ep,
      jnp.asarray(alpha, jnp.float32).reshape(1, 1),
      p["lW"], p["lb"].reshape(1, Dout),
      p["pW1"], p["pW2"], p["pb2"].reshape(1, H2), p["rW"])


def kernel(x, edge_index, alpha, k_ricci, e_poinc, params):
    src2 = edge_index[0].reshape(NCHUNK, K)
    dst2 = edge_index[1].reshape(NCHUNK, K)
    srcb = edge_index[0].reshape(NCHUNK_B, KB)
    dstb = edge_index[1].reshape(NCHUNK_B, KB)
    p1, p2 = params["l1"], params["l2"]

    e1 = _edge_weights(k_ricci, p1["hW1"], p1["hW2"], p1["hb2"])
    s1 = _sc_segment_sum_src_nocnt(e1, src2)
    cnt = _sc_count_dst(dst2)
    g1 = _gdiv(x, s1)
    acc1 = _sc_gather_mul_scatter(e1, g1, srcb, dstb)
    h = _finish(acc1, cnt, x, e_poinc, p1, alpha, last=False)

    e2 = _edge_weights(k_ricci, p2["hW1"], p2["hW2"], p2["hb2"])
    s2 = _sc_segment_sum_src_nocnt(e2, src2)
    g2 = _gdiv(h, s2)
    acc2 = _sc_gather_mul_scatter(e2, g2, srcb, dstb)
    return _finish(acc2, cnt, h, e_poinc, p2, alpha, last=True)
